# Initial kernel scaffold; baseline (speedup 1.0000x reference)
import jax
import jax.numpy as jnp
from jax.experimental import pallas as pl


def kernel(node_features, edge_index, etypes, W_edge, b_edge, gru_wi, gru_wh, gru_bi, gru_bh):
    raise NotImplementedError("write your pallas kernel here")


# speedup vs baseline: 11.5438x; 11.5438x over previous
import functools

import jax
import jax.numpy as jnp
from jax import lax
from jax.experimental import pallas as pl
from jax.experimental.pallas import tpu as pltpu
from jax.experimental.pallas import tpu_sc as plsc

STEPS = 3

NC = 2
NS = 16
CHUNK = 128


def _trans_body(h_ref, wt_ref, b_ref, out_ref):
    out_ref[0] = (
        jnp.dot(h_ref[...], wt_ref[0], preferred_element_type=jnp.float32)
        + b_ref[0]
    )


def _make_trans(N, H, T, BN):
    grid = (T, N // BN)
    return pl.pallas_call(
        _trans_body,
        grid=grid,
        in_specs=[
            pl.BlockSpec((BN, H), lambda t, i: (i, 0)),
            pl.BlockSpec((1, H, H), lambda t, i: (t, 0, 0)),
            pl.BlockSpec((1, 1, H), lambda t, i: (t, 0, 0)),
        ],
        out_specs=pl.BlockSpec((1, BN, H), lambda t, i: (t, i, 0)),
        out_shape=jax.ShapeDtypeStruct((T, N, H), jnp.float32),
    )


def _gru_body(ap_ref, h_ref, wit_ref, wht_ref, bi_ref, bh_ref, out_ref, *, H):
    a = ap_ref[0] + ap_ref[1]
    h = h_ref[...]
    gi = jnp.dot(a, wit_ref[...], preferred_element_type=jnp.float32) + bi_ref[...]
    gh = jnp.dot(h, wht_ref[...], preferred_element_type=jnp.float32) + bh_ref[...]
    r = jax.nn.sigmoid(gi[:, :H] + gh[:, :H])
    z = jax.nn.sigmoid(gi[:, H:2 * H] + gh[:, H:2 * H])
    n = jnp.tanh(gi[:, 2 * H:] + r * gh[:, 2 * H:])
    out_ref[...] = (1.0 - z) * n + z * h


def _make_gru(N, H, BN):
    grid = (N // BN,)
    return pl.pallas_call(
        functools.partial(_gru_body, H=H),
        grid=grid,
        in_specs=[
            pl.BlockSpec((2, BN, H), lambda i: (0, i, 0)),
            pl.BlockSpec((BN, H), lambda i: (i, 0)),
            pl.BlockSpec((H, 3 * H), lambda i: (0, 0)),
            pl.BlockSpec((H, 3 * H), lambda i: (0, 0)),
            pl.BlockSpec((1, 3 * H), lambda i: (0, 0)),
            pl.BlockSpec((1, 3 * H), lambda i: (0, 0)),
        ],
        out_specs=pl.BlockSpec((BN, H), lambda i: (i, 0)),
        out_shape=jax.ShapeDtypeStruct((N, H), jnp.float32),
    )


def _make_msgpass(N, H, N_pad, cpt):
    n_main = (N // NS) // 8 * 8
    n_last = N - n_main * (NS - 1)
    z_per_tile = N_pad // NS
    mesh = plsc.VectorSubcoreMesh(core_axis_name="c", subcore_axis_name="s")

    @functools.partial(
        pl.kernel,
        mesh=mesh,
        out_type=jax.ShapeDtypeStruct((NC, N, H), jnp.float32),
        scratch_types=[
            pltpu.VMEM((cpt, CHUNK), jnp.int32),
            pltpu.VMEM((cpt, CHUNK), jnp.int32),
            pltpu.VMEM((CHUNK, H), jnp.float32),
            pltpu.VMEM_SHARED((N_pad, H), jnp.float32),
            pltpu.SemaphoreType.DMA,
        ],
    )
    def msgpass(table_hbm, idx_hbm, dst_hbm, zeros_hbm, out_hbm,
                idx_v, dst_v, rows_v, acc, sem):
        c = lax.axis_index("c")
        s = lax.axis_index("s")
        wid = c * NS + s

        pltpu.sync_copy(zeros_hbm.at[pl.ds(s * z_per_tile, z_per_tile)],
                        acc.at[pl.ds(s * z_per_tile, z_per_tile)])

        row0 = wid * cpt
        pltpu.sync_copy(idx_hbm.at[pl.ds(row0, cpt)], idx_v)
        pltpu.sync_copy(dst_hbm.at[pl.ds(row0, cpt)], dst_v)

        plsc.subcore_barrier()

        def body(g, carry):
            pltpu.async_copy(table_hbm.at[idx_v.at[g]], rows_v, sem).wait()
            pltpu.sync_copy(rows_v, acc.at[dst_v.at[g]], add=True)
            return carry

        lax.fori_loop(0, cpt, body, 0)

        plsc.subcore_barrier()

        @pl.when(s < NS - 1)
        def _():
            pltpu.sync_copy(acc.at[pl.ds(s * n_main, n_main)],
                            out_hbm.at[c, pl.ds(s * n_main, n_main)])

        @pl.when(s == NS - 1)
        def _():
            pltpu.sync_copy(acc.at[pl.ds((NS - 1) * n_main, n_last)],
                            out_hbm.at[c, pl.ds((NS - 1) * n_main, n_last)])

    return msgpass


def kernel(node_features, edge_index, etypes, W_edge, b_edge,
           gru_wi, gru_wh, gru_bi, gru_bh):
    N, H = node_features.shape
    E = edge_index.shape[1]
    L, T = W_edge.shape[0], W_edge.shape[1]

    src = edge_index[0]
    dst = edge_index[1]

    workers = NC * NS
    per_tile = -(-E // workers)
    per_tile = -(-per_tile // (CHUNK * 8)) * (CHUNK * 8)
    cpt = per_tile // CHUNK
    e_pad = per_tile * workers
    n_pad = -(-(N + 1) // (NS * 8)) * (NS * 8)

    flat_idx = etypes * N + src
    pad = e_pad - E
    if pad:
        flat_idx = jnp.concatenate([flat_idx, jnp.zeros((pad,), jnp.int32)])
        dst_p = jnp.concatenate([dst, jnp.full((pad,), N, jnp.int32)])
    else:
        dst_p = dst
    idx2d = flat_idx.reshape(-1, CHUNK)
    dst2d = dst_p.reshape(-1, CHUNK)
    zeros_pad = jnp.zeros((n_pad, H), jnp.float32)

    W_T = jnp.transpose(W_edge, (0, 1, 3, 2))
    wiT = jnp.transpose(gru_wi, (0, 2, 1))
    whT = jnp.transpose(gru_wh, (0, 2, 1))
    bi2 = gru_bi[:, None, :]
    bh2 = gru_bh[:, None, :]

    trans_call = _make_trans(N, H, T, BN=2000)
    gru_call = _make_gru(N, H, BN=2000)
    msgpass_call = _make_msgpass(N, H, n_pad, cpt)

    h = node_features
    for l in range(L):
        for _ in range(STEPS):
            table = trans_call(h, W_T[l], b_edge[l][:, None, :]).reshape(T * N, H)
            parts = msgpass_call(table, idx2d, dst2d, zeros_pad)
            h = gru_call(parts, h, wiT[l], whT[l], bi2[l], bh2[l])
    return h


# speedup vs baseline: 12.7755x; 1.1067x over previous
import functools

import jax
import jax.numpy as jnp
from jax import lax
from jax.experimental import pallas as pl
from jax.experimental.pallas import tpu as pltpu
from jax.experimental.pallas import tpu_sc as plsc

STEPS = 3

NC = 2
NS = 16
CHUNK = 128


def _trans_body(h_ref, wt_ref, b_ref, out_ref):
    out_ref[0] = (
        jnp.dot(h_ref[...], wt_ref[0], preferred_element_type=jnp.float32)
        + b_ref[0]
    )


def _make_trans(N, H, T, BN):
    grid = (T, N // BN)
    return pl.pallas_call(
        _trans_body,
        grid=grid,
        in_specs=[
            pl.BlockSpec((BN, H), lambda t, i: (i, 0)),
            pl.BlockSpec((1, H, H), lambda t, i: (t, 0, 0)),
            pl.BlockSpec((1, 1, H), lambda t, i: (t, 0, 0)),
        ],
        out_specs=pl.BlockSpec((1, BN, H), lambda t, i: (t, i, 0)),
        out_shape=jax.ShapeDtypeStruct((T, N, H), jnp.float32),
    )


def _gru_body(ap_ref, h_ref, wit_ref, wht_ref, bi_ref, bh_ref, out_ref, *, H):
    a = ap_ref[0] + ap_ref[1]
    h = h_ref[...]
    gi = jnp.dot(a, wit_ref[...], preferred_element_type=jnp.float32) + bi_ref[...]
    gh = jnp.dot(h, wht_ref[...], preferred_element_type=jnp.float32) + bh_ref[...]
    r = jax.nn.sigmoid(gi[:, :H] + gh[:, :H])
    z = jax.nn.sigmoid(gi[:, H:2 * H] + gh[:, H:2 * H])
    n = jnp.tanh(gi[:, 2 * H:] + r * gh[:, 2 * H:])
    out_ref[...] = (1.0 - z) * n + z * h


def _make_gru(N, H, BN):
    grid = (N // BN,)
    return pl.pallas_call(
        functools.partial(_gru_body, H=H),
        grid=grid,
        in_specs=[
            pl.BlockSpec((2, BN, H), lambda i: (0, i, 0)),
            pl.BlockSpec((BN, H), lambda i: (i, 0)),
            pl.BlockSpec((H, 3 * H), lambda i: (0, 0)),
            pl.BlockSpec((H, 3 * H), lambda i: (0, 0)),
            pl.BlockSpec((1, 3 * H), lambda i: (0, 0)),
            pl.BlockSpec((1, 3 * H), lambda i: (0, 0)),
        ],
        out_specs=pl.BlockSpec((BN, H), lambda i: (i, 0)),
        out_shape=jax.ShapeDtypeStruct((N, H), jnp.float32),
    )


def _make_msgpass(N, H, N_pad, cpt):
    n_main = (N // NS) // 8 * 8
    n_last = N - n_main * (NS - 1)
    z_per_tile = N_pad // NS
    SB = 16
    n_sb = cpt // SB
    mesh = plsc.VectorSubcoreMesh(core_axis_name="c", subcore_axis_name="s")

    @functools.partial(
        pl.kernel,
        mesh=mesh,
        out_type=jax.ShapeDtypeStruct((NC, N, H), jnp.float32),
        scratch_types=[
            pltpu.VMEM((SB, CHUNK), jnp.int32),
            pltpu.VMEM((SB, CHUNK), jnp.int32),
            pltpu.VMEM((CHUNK, H), jnp.float32),
            pltpu.VMEM((CHUNK, H), jnp.float32),
            pltpu.VMEM_SHARED((N_pad, H), jnp.float32),
            pltpu.SemaphoreType.DMA,
            pltpu.SemaphoreType.DMA,
        ],
    )
    def msgpass(table_hbm, idx_hbm, dst_hbm, zeros_hbm, out_hbm,
                idx_v, dst_v, rows0_v, rows1_v, acc, sem0, sem1):
        c = lax.axis_index("c")
        s = lax.axis_index("s")
        wid = c * NS + s

        pltpu.sync_copy(zeros_hbm.at[pl.ds(s * z_per_tile, z_per_tile)],
                        acc.at[pl.ds(s * z_per_tile, z_per_tile)])

        row0 = wid * cpt
        plsc.subcore_barrier()

        def sb_body(b, carry):
            r0 = row0 + b * SB
            pltpu.sync_copy(idx_hbm.at[pl.ds(r0, SB)], idx_v)
            pltpu.sync_copy(dst_hbm.at[pl.ds(r0, SB)], dst_v)
            pltpu.async_copy(table_hbm.at[idx_v.at[0]], rows0_v, sem0)
            pltpu.async_copy(table_hbm.at[idx_v.at[1]], rows1_v, sem1)

            def body(i, c2):
                g = 2 * i
                pltpu.make_async_copy(table_hbm.at[idx_v.at[0]], rows0_v, sem0).wait()
                pltpu.sync_copy(rows0_v, acc.at[dst_v.at[g]], add=True)

                @pl.when(g + 2 < SB)
                def _():
                    pltpu.async_copy(table_hbm.at[idx_v.at[g + 2]], rows0_v, sem0)

                pltpu.make_async_copy(table_hbm.at[idx_v.at[1]], rows1_v, sem1).wait()
                pltpu.sync_copy(rows1_v, acc.at[dst_v.at[g + 1]], add=True)

                @pl.when(g + 3 < SB)
                def _():
                    pltpu.async_copy(table_hbm.at[idx_v.at[g + 3]], rows1_v, sem1)

                return c2

            lax.fori_loop(0, SB // 2, body, 0)
            return carry

        lax.fori_loop(0, n_sb, sb_body, 0)

        plsc.subcore_barrier()

        @pl.when(s < NS - 1)
        def _():
            pltpu.sync_copy(acc.at[pl.ds(s * n_main, n_main)],
                            out_hbm.at[c, pl.ds(s * n_main, n_main)])

        @pl.when(s == NS - 1)
        def _():
            pltpu.sync_copy(acc.at[pl.ds((NS - 1) * n_main, n_last)],
                            out_hbm.at[c, pl.ds((NS - 1) * n_main, n_last)])

    return msgpass


def kernel(node_features, edge_index, etypes, W_edge, b_edge,
           gru_wi, gru_wh, gru_bi, gru_bh):
    N, H = node_features.shape
    E = edge_index.shape[1]
    L, T = W_edge.shape[0], W_edge.shape[1]

    src = edge_index[0]
    dst = edge_index[1]

    workers = NC * NS
    per_tile = -(-E // workers)
    per_tile = -(-per_tile // (CHUNK * 8)) * (CHUNK * 8)
    cpt = per_tile // CHUNK
    e_pad = per_tile * workers
    n_pad = -(-(N + 1) // (NS * 8)) * (NS * 8)

    flat_idx = etypes * N + src
    pad = e_pad - E
    if pad:
        flat_idx = jnp.concatenate([flat_idx, jnp.zeros((pad,), jnp.int32)])
        dst_p = jnp.concatenate([dst, jnp.full((pad,), N, jnp.int32)])
    else:
        dst_p = dst
    idx2d = flat_idx.reshape(-1, CHUNK)
    dst2d = dst_p.reshape(-1, CHUNK)
    zeros_pad = jnp.zeros((n_pad, H), jnp.float32)

    W_T = jnp.transpose(W_edge, (0, 1, 3, 2))
    wiT = jnp.transpose(gru_wi, (0, 2, 1))
    whT = jnp.transpose(gru_wh, (0, 2, 1))
    bi2 = gru_bi[:, None, :]
    bh2 = gru_bh[:, None, :]

    trans_call = _make_trans(N, H, T, BN=2000)
    gru_call = _make_gru(N, H, BN=2000)
    msgpass_call = _make_msgpass(N, H, n_pad, cpt)

    h = node_features
    for l in range(L):
        for _ in range(STEPS):
            table = trans_call(h, W_T[l], b_edge[l][:, None, :]).reshape(T * N, H)
            parts = msgpass_call(table, idx2d, dst2d, zeros_pad)
            h = gru_call(parts, h, wiT[l], whT[l], bi2[l], bh2[l])
    return h
